# Initial kernel scaffold; baseline (speedup 1.0000x reference)
import jax
import jax.numpy as jnp
from jax.experimental import pallas as pl


def kernel(points, features, W1, g1, b1, Wa, ba, W2, g2, b2):
    raise NotImplementedError("write your pallas kernel here")


# speedup vs baseline: 9.7014x; 9.7014x over previous
import jax
import jax.numpy as jnp
from jax.experimental import pallas as pl

_R = 16
_NS = 2 * _R
_STRIDE = 2
_EPS = 1e-5
_OFFS = list(range(-_R, 0)) + list(range(1, _R + 1))


def _stage_a_body(f_ref, p_ref, w1_ref, w2_ref, z_ref, g_ref, sz_ref, szz_ref):
    f = f_ref[...]
    p = p_ref[...]
    w1 = w1_ref[...]
    z = jnp.dot(f, w2_ref[...], preferred_element_type=jnp.float32)
    gp = p[:, 0:1] * w1[0:1, :] + p[:, 1:2] * w1[1:2, :]
    g = gp + jnp.dot(f, w1[2:, :], preferred_element_type=jnp.float32)
    z_ref[...] = z
    g_ref[...] = g

    @pl.when(pl.program_id(0) == 0)
    def _():
        sz_ref[...] = jnp.zeros_like(sz_ref)
        szz_ref[...] = jnp.zeros_like(szz_ref)

    sz_ref[...] += jnp.sum(z, axis=0, keepdims=True)
    szz_ref[...] += jnp.sum(z * z, axis=0, keepdims=True)


def _stage_b_body(ge_ref, go_ref, pe_ref, w1_ref, sh_ref, shh_ref):
    w1 = w1_ref[...]
    pe = pe_ref[0]
    pwd = pe[:, 0:1] * w1[0:1, :] + pe[:, 1:2] * w1[1:2, :]
    ge = ge_ref[0]
    go = go_ref[0]

    @pl.when(pl.program_id(0) == 0)
    def _():
        sh_ref[...] = jnp.zeros_like(sh_ref)
        shh_ref[...] = jnp.zeros_like(shh_ref)

    nd = pe.shape[0]
    s = jnp.zeros((1, pwd.shape[1]), jnp.float32)
    ss = jnp.zeros((1, pwd.shape[1]), jnp.float32)
    for o in _OFFS:
        if o % 2 == 0:
            base = _R // 2 + o // 2
            h = ge[base:base + nd, :] - pwd
        else:
            base = _R // 2 + (o - 1) // 2
            h = go[base:base + nd, :] - pwd
        s += jnp.sum(h, axis=0, keepdims=True)
        ss += jnp.sum(h * h, axis=0, keepdims=True)
    sh_ref[...] += s
    shh_ref[...] += ss


def _stage_c_body(ge_ref, go_ref, ze_ref, zo_ref, pe_ref, w1_ref, wa_ref,
                  s1_ref, t1_ref, s2_ref, t2_ref, out_ref):
    w1 = w1_ref[...]
    pe = pe_ref[0]
    pwd = pe[:, 0:1] * w1[0:1, :] + pe[:, 1:2] * w1[1:2, :]
    ge = ge_ref[0]
    go = go_ref[0]
    ze = ze_ref[0]
    zo = zo_ref[0]
    wa = wa_ref[...]
    s1 = s1_ref[...]
    t1 = t1_ref[...]
    s2 = s2_ref[...]
    t2 = t2_ref[...]
    nd = pe.shape[0]

    logits = []
    for o in _OFFS:
        if o % 2 == 0:
            base = _R // 2 + o // 2
            h = ge[base:base + nd, :] - pwd
        else:
            base = _R // 2 + (o - 1) // 2
            h = go[base:base + nd, :] - pwd
        a = jnp.maximum(h * s1 + t1, 0.0)
        logits.append(jnp.dot(a, wa, preferred_element_type=jnp.float32))
    lg = jnp.concatenate(logits, axis=1)
    lg = lg - jnp.max(lg, axis=1, keepdims=True)
    e = jnp.exp(lg)
    w = e / jnp.sum(e, axis=1, keepdims=True)

    acc = jnp.zeros_like(out_ref[0])
    for j, o in enumerate(_OFFS):
        if o % 2 == 0:
            base = _R // 2 + o // 2
            zsl = ze[base:base + nd, :]
        else:
            base = _R // 2 + (o - 1) // 2
            zsl = zo[base:base + nd, :]
        yn = jnp.maximum(zsl * s2 + t2, 0.0)
        acc += w[:, j:j + 1] * yn
    out_ref[0] = acc


def kernel(points, features, W1, g1, b1, Wa, ba, W2, g2, b2):
    Bv, Nv, _ = points.shape
    C = features.shape[1]
    nrows = Bv * Nv
    nd = Nv // _STRIDE
    hal = _R // 2

    pts_flat = points.reshape(nrows, 2)

    n_tiles = 5
    tile = nrows // n_tiles
    z, g, sz, szz = pl.pallas_call(
        _stage_a_body,
        grid=(n_tiles,),
        in_specs=[
            pl.BlockSpec((tile, C), lambda i: (i, 0)),
            pl.BlockSpec((tile, 2), lambda i: (i, 0)),
            pl.BlockSpec(W1.shape, lambda i: (0, 0)),
            pl.BlockSpec(W2.shape, lambda i: (0, 0)),
        ],
        out_specs=[
            pl.BlockSpec((tile, C), lambda i: (i, 0)),
            pl.BlockSpec((tile, C), lambda i: (i, 0)),
            pl.BlockSpec((1, C), lambda i: (0, 0)),
            pl.BlockSpec((1, C), lambda i: (0, 0)),
        ],
        out_shape=[
            jax.ShapeDtypeStruct((nrows, C), jnp.float32),
            jax.ShapeDtypeStruct((nrows, C), jnp.float32),
            jax.ShapeDtypeStruct((1, C), jnp.float32),
            jax.ShapeDtypeStruct((1, C), jnp.float32),
        ],
    )(features, pts_flat, W1, W2)

    mu2 = sz / nrows
    var2 = szz / nrows - mu2 * mu2
    s2 = g2[None, :] / jnp.sqrt(var2 + _EPS)
    t2 = b2[None, :] - mu2 * s2

    g3 = g.reshape(Bv, Nv, C)
    z3 = z.reshape(Bv, Nv, C)

    def ext(x):
        return jnp.concatenate([x[:, -hal:], x, x[:, :hal]], axis=1)

    ge = ext(g3[:, 0::2])
    go = ext(g3[:, 1::2])
    ze = ext(z3[:, 0::2])
    zo = ext(z3[:, 1::2])
    pe = points[:, 0::2]

    next_specs = [
        pl.BlockSpec((1, nd + 2 * hal, C), lambda b: (b, 0, 0)),
        pl.BlockSpec((1, nd + 2 * hal, C), lambda b: (b, 0, 0)),
        pl.BlockSpec((1, nd, 2), lambda b: (b, 0, 0)),
        pl.BlockSpec(W1.shape, lambda b: (0, 0)),
    ]

    sh, shh = pl.pallas_call(
        _stage_b_body,
        grid=(Bv,),
        in_specs=next_specs,
        out_specs=[
            pl.BlockSpec((1, C), lambda b: (0, 0)),
            pl.BlockSpec((1, C), lambda b: (0, 0)),
        ],
        out_shape=[
            jax.ShapeDtypeStruct((1, C), jnp.float32),
            jax.ShapeDtypeStruct((1, C), jnp.float32),
        ],
    )(ge, go, pe, W1)

    cnt = Bv * nd * _NS
    mu1 = sh / cnt
    var1 = shh / cnt - mu1 * mu1
    s1 = g1[None, :] / jnp.sqrt(var1 + _EPS)
    t1 = b1[None, :] - mu1 * s1

    out = pl.pallas_call(
        _stage_c_body,
        grid=(Bv,),
        in_specs=[
            pl.BlockSpec((1, nd + 2 * hal, C), lambda b: (b, 0, 0)),
            pl.BlockSpec((1, nd + 2 * hal, C), lambda b: (b, 0, 0)),
            pl.BlockSpec((1, nd + 2 * hal, C), lambda b: (b, 0, 0)),
            pl.BlockSpec((1, nd + 2 * hal, C), lambda b: (b, 0, 0)),
            pl.BlockSpec((1, nd, 2), lambda b: (b, 0, 0)),
            pl.BlockSpec(W1.shape, lambda b: (0, 0)),
            pl.BlockSpec(Wa.shape, lambda b: (0, 0)),
            pl.BlockSpec((1, C), lambda b: (0, 0)),
            pl.BlockSpec((1, C), lambda b: (0, 0)),
            pl.BlockSpec((1, C), lambda b: (0, 0)),
            pl.BlockSpec((1, C), lambda b: (0, 0)),
        ],
        out_specs=pl.BlockSpec((1, nd, C), lambda b: (b, 0, 0)),
        out_shape=jax.ShapeDtypeStruct((Bv, nd, C), jnp.float32),
    )(ge, go, ze, zo, pe, W1, Wa, s1, t1, s2, t2)

    return (points[:, ::_STRIDE], out.reshape(Bv * nd, C))


# speedup vs baseline: 10.3509x; 1.0670x over previous
import jax
import jax.numpy as jnp
from jax.experimental import pallas as pl

_R = 16
_NS = 2 * _R
_STRIDE = 2
_EPS = 1e-5
_OFFS = list(range(-_R, 0)) + list(range(1, _R + 1))


def _slab(e_ref, o_ref, o, nd):
    if o % 2 == 0:
        base = _R // 2 + o // 2
        return e_ref[base:base + nd, :]
    base = _R // 2 + (o - 1) // 2
    return o_ref[base:base + nd, :]


def _stage_a_body(f_ref, p_ref, w1_ref, w2_ref,
                  z_ref, g_ref, sz_ref, szz_ref, sg_ref, sgg_ref):
    f = f_ref[...]
    p = p_ref[...]
    w1 = w1_ref[...]
    z = jnp.dot(f, w2_ref[...], preferred_element_type=jnp.float32)
    gp = p[:, 0:1] * w1[0:1, :] + p[:, 1:2] * w1[1:2, :]
    g = gp + jnp.dot(f, w1[2:, :], preferred_element_type=jnp.float32)
    z_ref[...] = z
    g_ref[...] = g

    @pl.when(pl.program_id(0) == 0)
    def _():
        sz_ref[...] = jnp.zeros_like(sz_ref)
        szz_ref[...] = jnp.zeros_like(szz_ref)
        sg_ref[...] = jnp.zeros_like(sg_ref)
        sgg_ref[...] = jnp.zeros_like(sgg_ref)

    sz_ref[...] += jnp.sum(z, axis=0, keepdims=True)
    szz_ref[...] += jnp.sum(z * z, axis=0, keepdims=True)
    sg_ref[...] += jnp.sum(g, axis=0, keepdims=True)
    sgg_ref[...] += jnp.sum(g * g, axis=0, keepdims=True)


def _stage_b_body(ge_ref, go_ref, pe_ref, w1_ref,
                  cross_ref, spw_ref, spw2_ref):
    w1 = w1_ref[...]
    pe = pe_ref[0]
    pwd = pe[:, 0:1] * w1[0:1, :] + pe[:, 1:2] * w1[1:2, :]
    ge = ge_ref[0]
    go = go_ref[0]

    @pl.when(pl.program_id(0) == 0)
    def _():
        cross_ref[...] = jnp.zeros_like(cross_ref)
        spw_ref[...] = jnp.zeros_like(spw_ref)
        spw2_ref[...] = jnp.zeros_like(spw2_ref)

    nd = pe.shape[0]
    s = _slab(ge, go, _OFFS[0], nd)
    for o in _OFFS[1:]:
        s = s + _slab(ge, go, o, nd)
    cross_ref[...] += jnp.sum(pwd * s, axis=0, keepdims=True)
    spw_ref[...] += jnp.sum(pwd, axis=0, keepdims=True)
    spw2_ref[...] += jnp.sum(pwd * pwd, axis=0, keepdims=True)


def _stage_c_body(ge_ref, go_ref, ze_ref, zo_ref, pe_ref, w1_ref, wa_ref,
                  s1_ref, t1_ref, s2_ref, t2_ref, out_ref):
    w1 = w1_ref[...]
    pe = pe_ref[0]
    s1 = s1_ref[...]
    t1 = t1_ref[...]
    s2 = s2_ref[...]
    t2 = t2_ref[...]
    wa = wa_ref[...]
    nd = pe.shape[0]

    pwd = pe[:, 0:1] * w1[0:1, :] + pe[:, 1:2] * w1[1:2, :]
    qd = t1 - pwd * s1
    gse = ge_ref[0] * s1
    gso = go_ref[0] * s1
    yne = jnp.maximum(ze_ref[0] * s2 + t2, 0.0)
    yno = jnp.maximum(zo_ref[0] * s2 + t2, 0.0)

    logits = []
    for o in _OFFS:
        a = jnp.maximum(_slab(gse, gso, o, nd) + qd, 0.0)
        logits.append(jnp.dot(a, wa, preferred_element_type=jnp.float32))
    lg = jnp.concatenate(logits, axis=1)
    lg = lg - jnp.max(lg, axis=1, keepdims=True)
    e = jnp.exp(lg)
    w = e / jnp.sum(e, axis=1, keepdims=True)

    acc = jnp.zeros_like(out_ref[0])
    for j, o in enumerate(_OFFS):
        acc += w[:, j:j + 1] * _slab(yne, yno, o, nd)
    out_ref[0] = acc


def kernel(points, features, W1, g1, b1, Wa, ba, W2, g2, b2):
    Bv, Nv, _ = points.shape
    C = features.shape[1]
    nrows = Bv * Nv
    nd = Nv // _STRIDE
    hal = _R // 2

    pts_flat = points.reshape(nrows, 2)

    n_tiles = 5
    tile = nrows // n_tiles
    z, g, sz, szz, sg, sgg = pl.pallas_call(
        _stage_a_body,
        grid=(n_tiles,),
        in_specs=[
            pl.BlockSpec((tile, C), lambda i: (i, 0)),
            pl.BlockSpec((tile, 2), lambda i: (i, 0)),
            pl.BlockSpec(W1.shape, lambda i: (0, 0)),
            pl.BlockSpec(W2.shape, lambda i: (0, 0)),
        ],
        out_specs=[
            pl.BlockSpec((tile, C), lambda i: (i, 0)),
            pl.BlockSpec((tile, C), lambda i: (i, 0)),
            pl.BlockSpec((1, C), lambda i: (0, 0)),
            pl.BlockSpec((1, C), lambda i: (0, 0)),
            pl.BlockSpec((1, C), lambda i: (0, 0)),
            pl.BlockSpec((1, C), lambda i: (0, 0)),
        ],
        out_shape=[
            jax.ShapeDtypeStruct((nrows, C), jnp.float32),
            jax.ShapeDtypeStruct((nrows, C), jnp.float32),
            jax.ShapeDtypeStruct((1, C), jnp.float32),
            jax.ShapeDtypeStruct((1, C), jnp.float32),
            jax.ShapeDtypeStruct((1, C), jnp.float32),
            jax.ShapeDtypeStruct((1, C), jnp.float32),
        ],
    )(features, pts_flat, W1, W2)

    mu2 = sz / nrows
    var2 = szz / nrows - mu2 * mu2
    s2 = g2[None, :] / jnp.sqrt(var2 + _EPS)
    t2 = b2[None, :] - mu2 * s2

    g3 = g.reshape(Bv, Nv, C)
    z3 = z.reshape(Bv, Nv, C)

    def ext(x):
        return jnp.concatenate([x[:, -hal:], x, x[:, :hal]], axis=1)

    ge = ext(g3[:, 0::2])
    go = ext(g3[:, 1::2])
    ze = ext(z3[:, 0::2])
    zo = ext(z3[:, 1::2])
    pe = points[:, 0::2]

    cross, spw, spw2 = pl.pallas_call(
        _stage_b_body,
        grid=(Bv,),
        in_specs=[
            pl.BlockSpec((1, nd + 2 * hal, C), lambda b: (b, 0, 0)),
            pl.BlockSpec((1, nd + 2 * hal, C), lambda b: (b, 0, 0)),
            pl.BlockSpec((1, nd, 2), lambda b: (b, 0, 0)),
            pl.BlockSpec(W1.shape, lambda b: (0, 0)),
        ],
        out_specs=[
            pl.BlockSpec((1, C), lambda b: (0, 0)),
            pl.BlockSpec((1, C), lambda b: (0, 0)),
            pl.BlockSpec((1, C), lambda b: (0, 0)),
        ],
        out_shape=[
            jax.ShapeDtypeStruct((1, C), jnp.float32),
            jax.ShapeDtypeStruct((1, C), jnp.float32),
            jax.ShapeDtypeStruct((1, C), jnp.float32),
        ],
    )(ge, go, pe, W1)

    cnt = Bv * nd * _NS
    mult = _NS // _STRIDE
    sum_h = mult * sg - _NS * spw
    ssq_h = mult * sgg - 2.0 * cross + _NS * spw2
    mu1 = sum_h / cnt
    var1 = ssq_h / cnt - mu1 * mu1
    s1 = g1[None, :] / jnp.sqrt(var1 + _EPS)
    t1 = b1[None, :] - mu1 * s1

    out = pl.pallas_call(
        _stage_c_body,
        grid=(Bv,),
        in_specs=[
            pl.BlockSpec((1, nd + 2 * hal, C), lambda b: (b, 0, 0)),
            pl.BlockSpec((1, nd + 2 * hal, C), lambda b: (b, 0, 0)),
            pl.BlockSpec((1, nd + 2 * hal, C), lambda b: (b, 0, 0)),
            pl.BlockSpec((1, nd + 2 * hal, C), lambda b: (b, 0, 0)),
            pl.BlockSpec((1, nd, 2), lambda b: (b, 0, 0)),
            pl.BlockSpec(W1.shape, lambda b: (0, 0)),
            pl.BlockSpec(Wa.shape, lambda b: (0, 0)),
            pl.BlockSpec((1, C), lambda b: (0, 0)),
            pl.BlockSpec((1, C), lambda b: (0, 0)),
            pl.BlockSpec((1, C), lambda b: (0, 0)),
            pl.BlockSpec((1, C), lambda b: (0, 0)),
        ],
        out_specs=pl.BlockSpec((1, nd, C), lambda b: (b, 0, 0)),
        out_shape=jax.ShapeDtypeStruct((Bv, nd, C), jnp.float32),
    )(ge, go, ze, zo, pe, W1, Wa, s1, t1, s2, t2)

    return (points[:, ::_STRIDE], out.reshape(Bv * nd, C))


# speedup vs baseline: 17.9124x; 1.7305x over previous
import jax
import jax.numpy as jnp
from jax.experimental import pallas as pl
from jax.experimental.pallas import tpu as pltpu

_R = 16
_NS = 2 * _R
_STRIDE = 2
_EPS = 1e-5
_OFFS = list(range(-_R, 0)) + list(range(1, _R + 1))


def _slab(even, odd, o, nd):
    if o % 2 == 0:
        base = _R // 2 + o // 2
        return even[base:base + nd, :]
    base = _R // 2 + (o - 1) // 2
    return odd[base:base + nd, :]


def _ext(x, hal):
    n = x.shape[0]
    return jnp.concatenate([x[n - hal:], x, x[:hal]], axis=0)


def _pw(p, w1):
    return p[:, 0:1] * w1[0:1, :] + p[:, 1:2] * w1[1:2, :]


def _fused_body(fe_ref, fo_ref, pe_ref, po_ref, w1_ref, wa_ref,
                g1_ref, b1_ref, g2_ref, b2_ref, w2_ref, out_ref,
                ge_s, go_s, ze_s, zo_s, acc_s, st_s):
    ph = pl.program_id(0)
    b = pl.program_id(1)
    Bv = ge_s.shape[0]
    nd = fe_ref.shape[1]
    C = fe_ref.shape[2]
    hal = _R // 2
    w1 = w1_ref[...]

    @pl.when(ph == 0)
    def _phase0():
        fe = fe_ref[0]
        fo = fo_ref[0]
        pwe_b = _pw(pe_ref[0], w1)
        pwo_b = _pw(po_ref[0], w1)
        w1b = w1[2:, :]
        w2 = w2_ref[...]
        ge = pwe_b + jnp.dot(fe, w1b, preferred_element_type=jnp.float32)
        go = pwo_b + jnp.dot(fo, w1b, preferred_element_type=jnp.float32)
        ze = jnp.dot(fe, w2, preferred_element_type=jnp.float32)
        zo = jnp.dot(fo, w2, preferred_element_type=jnp.float32)
        ge_s[b] = ge
        go_s[b] = go
        ze_s[b] = ze
        zo_s[b] = zo

        @pl.when(b == 0)
        def _():
            acc_s[...] = jnp.zeros_like(acc_s)

        acc_s[0:1] += jnp.sum(ze, axis=0, keepdims=True) + jnp.sum(zo, axis=0, keepdims=True)
        acc_s[1:2] += jnp.sum(ze * ze, axis=0, keepdims=True) + jnp.sum(zo * zo, axis=0, keepdims=True)
        acc_s[2:3] += jnp.sum(ge, axis=0, keepdims=True) + jnp.sum(go, axis=0, keepdims=True)
        acc_s[3:4] += jnp.sum(ge * ge, axis=0, keepdims=True) + jnp.sum(go * go, axis=0, keepdims=True)

        ge_b = _ext(ge, hal)
        go_b = _ext(go, hal)
        s = _slab(ge_b, go_b, _OFFS[0], nd)
        for o in _OFFS[1:]:
            s = s + _slab(ge_b, go_b, o, nd)
        acc_s[4:5] += jnp.sum(pwe_b * s, axis=0, keepdims=True)
        acc_s[5:6] += jnp.sum(pwe_b, axis=0, keepdims=True)
        acc_s[6:7] += jnp.sum(pwe_b * pwe_b, axis=0, keepdims=True)

    @pl.when(ph == 1)
    def _phase1():

        @pl.when(b == 0)
        def _():
            nrows = Bv * nd * 2
            mu2 = acc_s[0:1] / nrows
            var2 = acc_s[1:2] / nrows - mu2 * mu2
            s2 = g2_ref[...] / jnp.sqrt(var2 + _EPS)
            t2 = b2_ref[...] - mu2 * s2
            cnt = Bv * nd * _NS
            mult = _NS // _STRIDE
            sum_h = mult * acc_s[2:3] - _NS * acc_s[5:6]
            ssq_h = mult * acc_s[3:4] - 2.0 * acc_s[4:5] + _NS * acc_s[6:7]
            mu1 = sum_h / cnt
            var1 = ssq_h / cnt - mu1 * mu1
            s1 = g1_ref[...] / jnp.sqrt(var1 + _EPS)
            t1 = b1_ref[...] - mu1 * s1
            st_s[0:1] = s1
            st_s[1:2] = t1
            st_s[2:3] = s2
            st_s[3:4] = t2

        s1 = st_s[0:1]
        t1 = st_s[1:2]
        s2 = st_s[2:3]
        t2 = st_s[3:4]
        wa = wa_ref[...]

        qd = t1 - _pw(pe_ref[0], w1) * s1
        gse = _ext(ge_s[b] * s1, hal)
        gso = _ext(go_s[b] * s1, hal)
        yne = _ext(jnp.maximum(ze_s[b] * s2 + t2, 0.0), hal)
        yno = _ext(jnp.maximum(zo_s[b] * s2 + t2, 0.0), hal)

        logits = []
        for o in _OFFS:
            a = jnp.maximum(_slab(gse, gso, o, nd) + qd, 0.0)
            logits.append(jnp.dot(a, wa, preferred_element_type=jnp.float32))
        lg = jnp.concatenate(logits, axis=1)
        lg = lg - jnp.max(lg, axis=1, keepdims=True)
        e = jnp.exp(lg)
        w = e / jnp.sum(e, axis=1, keepdims=True)

        acc = w[:, 0:1] * _slab(yne, yno, _OFFS[0], nd)
        for j, o in enumerate(_OFFS[1:]):
            acc += w[:, j + 1:j + 2] * _slab(yne, yno, o, nd)
        out_ref[0] = acc


def kernel(points, features, W1, g1, b1, Wa, ba, W2, g2, b2):
    Bv, Nv, _ = points.shape
    C = features.shape[1]
    nd = Nv // _STRIDE

    f3 = features.reshape(Bv, Nv, C)
    fe = f3[:, 0::2]
    fo = f3[:, 1::2]
    pe = points[:, 0::2]
    po = points[:, 1::2]

    bmap = lambda ph, b: (b, 0, 0)
    pmap = lambda ph, b: (b, 0, 0)
    cmap2 = lambda ph, b: (0, 0)

    out = pl.pallas_call(
        _fused_body,
        grid=(2, Bv),
        in_specs=[
            pl.BlockSpec((1, nd, C), bmap),
            pl.BlockSpec((1, nd, C), bmap),
            pl.BlockSpec((1, nd, 2), pmap),
            pl.BlockSpec((1, nd, 2), bmap),
            pl.BlockSpec(W1.shape, cmap2),
            pl.BlockSpec(Wa.shape, cmap2),
            pl.BlockSpec((1, C), cmap2),
            pl.BlockSpec((1, C), cmap2),
            pl.BlockSpec((1, C), cmap2),
            pl.BlockSpec((1, C), cmap2),
            pl.BlockSpec(W2.shape, cmap2),
        ],
        out_specs=pl.BlockSpec((1, nd, C), pmap),
        out_shape=jax.ShapeDtypeStruct((Bv, nd, C), jnp.float32),
        scratch_shapes=[
            pltpu.VMEM((Bv, nd, C), jnp.float32),
            pltpu.VMEM((Bv, nd, C), jnp.float32),
            pltpu.VMEM((Bv, nd, C), jnp.float32),
            pltpu.VMEM((Bv, nd, C), jnp.float32),
            pltpu.VMEM((7, C), jnp.float32),
            pltpu.VMEM((4, C), jnp.float32),
        ],
    )(fe, fo, pe, po, W1, Wa,
      g1.reshape(1, C), b1.reshape(1, C), g2.reshape(1, C), b2.reshape(1, C),
      W2)

    return (pe, out.reshape(Bv * nd, C))
